# baseline (speedup 1.0000x reference)
import functools

import jax
import jax.numpy as jnp
from jax import lax
from jax.experimental import pallas as pl
from jax.experimental.pallas import tpu as pltpu
from jax.experimental.pallas import tpu_sc as plsc

NUM_CORES = 2
NUM_SUBCORES = 16
LANES = 16
NW = NUM_CORES * NUM_SUBCORES
CHUNK = 25088
FSEG = 4096


def _sc_partials(label, feat_t, centers_t, tailp):
    d_dim, b = feat_t.shape
    _, v = centers_t.shape
    d_per_w = d_dim // NW
    main_w = (v // 128) * 128
    row_pad = main_w + 128
    n_chunks = -(-main_w // CHUNK)
    offs = [c * CHUNK for c in range(n_chunks)]
    lens = [min(CHUNK, main_w - o) for o in offs]
    n_segs = b // FSEG
    n_vec = FSEG // LANES
    mesh = plsc.VectorSubcoreMesh(core_axis_name="c", subcore_axis_name="s")

    @functools.partial(
        pl.kernel,
        mesh=mesh,
        out_type=jax.ShapeDtypeStruct((NW, 128), jnp.float32),
        scratch_types=[
            pltpu.VMEM((b,), jnp.int32),
            pltpu.VMEM_SHARED((b,), jnp.int32),
            pltpu.VMEM((FSEG,), jnp.float32),
            pltpu.VMEM((FSEG,), jnp.float32),
            pltpu.VMEM((row_pad,), jnp.float32),
            pltpu.VMEM((128,), jnp.float32),
            pltpu.SemaphoreType.DMA,
            pltpu.SemaphoreType.DMA,
            pltpu.SemaphoreType.DMA,
            pltpu.SemaphoreType.DMA,
        ],
        compiler_params=pltpu.CompilerParams(
            use_tc_tiling_on_sc=True, needs_layout_passes=False),
    )
    def k(label_hbm, feat_hbm, centers_hbm, tailp_hbm, out_hbm, lab_v,
          labsh_v, frow0_v, frow1_v, row_v, part_v, lab_sem, frow0_sem,
          frow1_sem, row_sem):
        sid = lax.axis_index("s")
        wid = sid * NUM_CORES + lax.axis_index("c")
        lab_chunk = b // NUM_SUBCORES
        frow_v = [frow0_v, frow1_v]
        frow_sem = [frow0_sem, frow1_sem]
        segs = [(di, q) for di in range(d_per_w) for q in range(n_segs)]

        def start_row(di):
            d = wid * d_per_w + di
            cps = [
                pltpu.async_copy(
                    centers_hbm.at[d, pl.ds(offs[c], lens[c])],
                    row_v.at[pl.ds(offs[c], lens[c])],
                    row_sem,
                )
                for c in range(n_chunks)
            ]
            cps.append(pltpu.async_copy(
                tailp_hbm.at[d], row_v.at[pl.ds(main_w, 128)], row_sem))
            return cps

        def start_frow(s, buf):
            di, q = segs[s]
            return pltpu.async_copy(
                feat_hbm.at[wid * d_per_w + di, pl.ds(q * FSEG, FSEG)],
                frow_v[buf], frow_sem[buf])

        lab_cp = pltpu.async_copy(
            label_hbm.at[pl.ds(sid * lab_chunk, lab_chunk)],
            labsh_v.at[pl.ds(sid * lab_chunk, lab_chunk)], lab_sem)
        row_cps = start_row(0)
        fpend = start_frow(0, 0)
        lab_cp.wait()
        plsc.subcore_barrier()
        pltpu.sync_copy(labsh_v, lab_v)

        acc = jnp.zeros((LANES,), jnp.float32)
        for s, (di, q) in enumerate(segs):
            buf = s % 2
            if q == 0:
                for cp in row_cps:
                    cp.wait()
            fpend.wait()
            if s + 1 < len(segs):
                fpend = start_frow(s + 1, 1 - buf)
            frow = frow_v[buf]

            def body(i, a, _base=q * FSEG, _frow=frow):
                st = i * LANES
                lab = lab_v[pl.ds(_base + st, LANES)]
                cval = plsc.load_gather(row_v, [lab])
                fval = _frow[pl.ds(st, LANES)]
                diff = fval - cval
                return a + diff * diff

            acc = lax.fori_loop(0, n_vec, body, acc, unroll=8)
            if q == n_segs - 1 and di + 1 < d_per_w:
                row_cps = start_row(di + 1)

        zero = jnp.zeros((LANES,), jnp.float32)
        for j in range(128 // LANES):
            part_v[pl.ds(j * LANES, LANES)] = zero
        part_v[pl.ds(0, LANES)] = acc
        pltpu.sync_copy(part_v, out_hbm.at[wid])

    return k(label, feat_t, centers_t, tailp)


def _reduce_partials(partials, scale):
    def rk(p_ref, o_ref):
        o_ref[0, 0] = jnp.sum(p_ref[...]) * scale

    return pl.pallas_call(
        rk,
        out_shape=jax.ShapeDtypeStruct((1, 1), jnp.float32),
        out_specs=pl.BlockSpec(memory_space=pltpu.SMEM),
    )(partials)


def kernel(label, feat, centers):
    batch = feat.shape[0]
    feat = feat.reshape(batch, -1)
    centers_t = centers.T
    v = centers_t.shape[1]
    main_w = (v // 128) * 128
    tailp = jnp.pad(centers_t[:, main_w:], ((0, 0), (0, 128 - (v - main_w))))
    partials = _sc_partials(label.astype(jnp.int32), feat.T, centers_t, tailp)
    out = _reduce_partials(partials, 0.5 / batch)
    return out[0, 0]
